# baseline (speedup 1.0000x reference)
import jax
import jax.numpy as jnp
from jax import lax
from jax.experimental import pallas as pl
from jax.experimental.pallas import tpu as pltpu
from jax.experimental.pallas import tpu_sc as plsc

NUM_INPUTS = 512
NUM_COMPUTED = 4096
NUM_OUTPUTS = 128
FAN_IN = 64
SCALE = 4.9
N_UNITS = NUM_INPUTS + 1 + NUM_COMPUTED
CARRY_PAD = 4624
CHUNK = 512
N_CHUNKS = NUM_COMPUTED // CHUNK
GROUPS = CHUNK // 16
N_GROUPS = NUM_COMPUTED // 16
OUT_BASE = NUM_INPUTS + 1 + (NUM_COMPUTED - NUM_OUTPUTS)


def _body(x_hbm, w_hbm, idx_hbm, mg_hbm, out_hbm, carry, w_v, idx_v, mg_v, st):
    wid = lax.axis_index("s") * 2 + lax.axis_index("c")

    @pl.when(wid == 0)
    def _():
        lane = jnp.arange(16, dtype=jnp.int32)
        ones = jnp.ones((16,), jnp.float32)

        pltpu.sync_copy(mg_hbm, mg_v)
        pltpu.sync_copy(x_hbm, carry.at[pl.ds(0, NUM_INPUTS)])

        def init_ones(i, _):
            carry[pl.ds(NUM_INPUTS + 16 * i, 16)] = ones
            return _

        lax.fori_loop(0, (CARRY_PAD - NUM_INPUTS) // 16, init_ones, 0)

        def group_step(g, state):
            pos, gg = state
            goff = g * (16 * FAN_IN)
            posv = pos + lane

            nacc = 4
            accs = [jnp.zeros((16,), jnp.float32) for _ in range(nacc)]
            for k in range(FAN_IN):
                iv = idx_v[pl.ds(goff + 16 * k, 16)]
                wv = w_v[pl.ds(goff + 16 * k, 16)]
                vals = plsc.load_gather(carry, [iv])
                wm = jnp.where(iv < pos, wv, 0.0)
                accs[k % nacc] = accs[k % nacc] + vals * wm
            acc_ext = (accs[0] + accs[1]) + (accs[2] + accs[3])

            val = 1.0 / (1.0 + jnp.exp(-SCALE * acc_ext))
            plsc.store_scatter(carry, [posv], val)

            gbase = (gg // 16) * 16
            mgs = mg_v[pl.ds(gbase, 16)]
            m_g = jnp.sum(jnp.where(lane == gg - gbase, mgs, 0))
            tail0 = goff + 16 * FAN_IN - 16 * m_g

            def fix_body(d):
                vcur = plsc.load_gather(carry, [posv])

                def tail_term(j, a):
                    iv = idx_v[pl.ds(tail0 + 16 * j, 16)]
                    wv = w_v[pl.ds(tail0 + 16 * j, 16)]
                    vals = plsc.load_gather(carry, [iv])
                    vals = jnp.where(iv < posv, vals, 1.0)
                    return a + jnp.where(iv >= pos, vals * wv, 0.0)

                acc = lax.fori_loop(0, m_g, tail_term, acc_ext)
                vnew = 1.0 / (1.0 + jnp.exp(-SCALE * acc))
                plsc.store_scatter(carry, [posv], vnew)
                return jnp.sum((vnew != vcur).astype(jnp.int32))

            lax.while_loop(lambda d: d > 0, fix_body, m_g)
            return pos + 16, gg + 1

        def chunk_step(c, state):
            off = c * (CHUNK * FAN_IN)
            pltpu.sync_copy(w_hbm.at[pl.ds(off, CHUNK * FAN_IN)], w_v)
            pltpu.sync_copy(idx_hbm.at[pl.ds(off, CHUNK * FAN_IN)], idx_v)
            return lax.fori_loop(0, GROUPS, group_step, state)

        lax.fori_loop(0, N_CHUNKS, chunk_step, (NUM_INPUTS + 1, 0))

        for i in range(NUM_OUTPUTS // 16):
            iv = jnp.full((16,), OUT_BASE + 16 * i, jnp.int32) + lane
            st[pl.ds(16 * i, 16)] = plsc.load_gather(carry, [iv])
        pltpu.sync_copy(st, out_hbm)


@jax.jit
def kernel(x, W, input_ids):
    mesh = plsc.VectorSubcoreMesh(core_axis_name="c", subcore_axis_name="s")
    run = pl.kernel(
        _body,
        out_type=jax.ShapeDtypeStruct((NUM_OUTPUTS,), jnp.float32),
        mesh=mesh,
        scratch_types=[
            pltpu.VMEM((CARRY_PAD,), jnp.float32),
            pltpu.VMEM((CHUNK * FAN_IN,), jnp.float32),
            pltpu.VMEM((CHUNK * FAN_IN,), jnp.int32),
            pltpu.VMEM((N_GROUPS,), jnp.int32),
            pltpu.VMEM((NUM_OUTPUTS,), jnp.float32),
        ],
        compiler_params=pltpu.CompilerParams(needs_layout_passes=False),
    )
    internal = input_ids >= (
        NUM_INPUTS + 1 + (jnp.arange(NUM_COMPUTED, dtype=jnp.int32) // 16 * 16)
    )[:, None]
    ii = internal.astype(jnp.int32)
    cs = jnp.cumsum(ii, axis=1)
    n_int = cs[:, -1]
    rank = jnp.where(internal, (FAN_IN - n_int[:, None]) + (cs - ii),
                     jnp.arange(FAN_IN, dtype=jnp.int32)[None, :] - cs)
    u = jnp.arange(NUM_COMPUTED, dtype=jnp.int32)[:, None]
    dest = (u // 16) * (16 * FAN_IN) + rank * 16 + (u % 16)
    df = dest.reshape(-1)
    wT = jnp.zeros((NUM_COMPUTED * FAN_IN,), jnp.float32).at[df].set(
        W.reshape(-1), unique_indices=True)
    idxT = jnp.zeros((NUM_COMPUTED * FAN_IN,), jnp.int32).at[df].set(
        input_ids.reshape(-1), unique_indices=True)
    mg = jnp.max(n_int.reshape(N_GROUPS, 16), axis=1)
    out = run(x.reshape(-1), wT, idxT, mg)
    return out[None, :]


# speedup vs baseline: 11.2625x; 11.2625x over previous
import jax
import jax.numpy as jnp
from jax import lax
from jax.experimental import pallas as pl
from jax.experimental.pallas import tpu as pltpu
from jax.experimental.pallas import tpu_sc as plsc

NUM_INPUTS = 512
NUM_COMPUTED = 4096
NUM_OUTPUTS = 128
FAN_IN = 64
SCALE = 4.9
N_UNITS = NUM_INPUTS + 1 + NUM_COMPUTED
CARRY_PAD = 4624
CHUNK = 256
N_CHUNKS = NUM_COMPUTED // CHUNK
GROUPS = CHUNK // 16
N_GROUPS = NUM_COMPUTED // 16
OUT_BASE = NUM_INPUTS + 1 + (NUM_COMPUTED - NUM_OUTPUTS)
CELEMS = CHUNK * FAN_IN


def _body(x_hbm, w_hbm, idx_hbm, mg_hbm, out_hbm,
          carry, w_a, idx_a, w_b, idx_b, mg_v, st,
          sem_wa, sem_ia, sem_wb, sem_ib):
    wid = lax.axis_index("s") * 2 + lax.axis_index("c")

    @pl.when(wid == 0)
    def _():
        lane = jnp.arange(16, dtype=jnp.int32)
        ones = jnp.ones((16,), jnp.float32)

        def start_load(c, w_buf, idx_buf, w_sem, i_sem):
            off = c * CELEMS
            pltpu.make_async_copy(
                w_hbm.at[pl.ds(off, CELEMS)], w_buf, w_sem).start()
            pltpu.make_async_copy(
                idx_hbm.at[pl.ds(off, CELEMS)], idx_buf, i_sem).start()

        def wait_load(w_buf, idx_buf, w_sem, i_sem):
            pltpu.make_async_copy(
                w_hbm.at[pl.ds(0, CELEMS)], w_buf, w_sem).wait()
            pltpu.make_async_copy(
                idx_hbm.at[pl.ds(0, CELEMS)], idx_buf, i_sem).wait()

        start_load(0, w_a, idx_a, sem_wa, sem_ia)
        pltpu.sync_copy(mg_hbm, mg_v)
        pltpu.sync_copy(x_hbm, carry.at[pl.ds(0, NUM_INPUTS)])

        def init_ones(i, _):
            carry[pl.ds(NUM_INPUTS + 16 * i, 16)] = ones
            return _

        lax.fori_loop(0, (CARRY_PAD - NUM_INPUTS) // 16, init_ones, 0)

        def make_group_step(w_v, idx_v):
            def group_step(g, state):
                pos, gg = state
                goff = g * (16 * FAN_IN)
                posv = pos + lane

                nacc = 4
                accs = [jnp.zeros((16,), jnp.float32) for _ in range(nacc)]
                for k in range(FAN_IN):
                    iv = idx_v[pl.ds(goff + 16 * k, 16)]
                    wv = w_v[pl.ds(goff + 16 * k, 16)]
                    vals = plsc.load_gather(carry, [iv])
                    wm = jnp.where(iv < pos, wv, 0.0)
                    accs[k % nacc] = accs[k % nacc] + vals * wm
                acc_ext = (accs[0] + accs[1]) + (accs[2] + accs[3])

                val = 1.0 / (1.0 + jnp.exp(-SCALE * acc_ext))
                plsc.store_scatter(carry, [posv], val)

                gbase = (gg // 16) * 16
                mgs = mg_v[pl.ds(gbase, 16)]
                m_g = jnp.sum(jnp.where(lane == gg - gbase, mgs, 0))
                tail0 = goff + 16 * FAN_IN - 16 * m_g

                def fix_body(d):
                    vcur = plsc.load_gather(carry, [posv])

                    def tail_term(j, a):
                        iv = idx_v[pl.ds(tail0 + 16 * j, 16)]
                        wv = w_v[pl.ds(tail0 + 16 * j, 16)]
                        vals = plsc.load_gather(carry, [iv])
                        vals = jnp.where(iv < posv, vals, 1.0)
                        return a + jnp.where(iv >= pos, vals * wv, 0.0)

                    acc = lax.fori_loop(0, m_g, tail_term, acc_ext)
                    vnew = 1.0 / (1.0 + jnp.exp(-SCALE * acc))
                    plsc.store_scatter(carry, [posv], vnew)
                    return jnp.sum((vnew != vcur).astype(jnp.int32))

                lax.while_loop(lambda d: d > 0, fix_body, m_g)
                return pos + 16, gg + 1

            return group_step

        step_a = make_group_step(w_a, idx_a)
        step_b = make_group_step(w_b, idx_b)

        def pair_step(p, state):
            c = 2 * p
            wait_load(w_a, idx_a, sem_wa, sem_ia)
            start_load(c + 1, w_b, idx_b, sem_wb, sem_ib)
            state = lax.fori_loop(0, GROUPS, step_a, state)
            wait_load(w_b, idx_b, sem_wb, sem_ib)

            @pl.when(p + 1 < N_CHUNKS // 2)
            def _():
                start_load(c + 2, w_a, idx_a, sem_wa, sem_ia)

            return lax.fori_loop(0, GROUPS, step_b, state)

        lax.fori_loop(0, N_CHUNKS // 2, pair_step, (NUM_INPUTS + 1, 0))

        for i in range(NUM_OUTPUTS // 16):
            iv = jnp.full((16,), OUT_BASE + 16 * i, jnp.int32) + lane
            st[pl.ds(16 * i, 16)] = plsc.load_gather(carry, [iv])
        pltpu.sync_copy(st, out_hbm)


@jax.jit
def kernel(x, W, input_ids):
    mesh = plsc.VectorSubcoreMesh(core_axis_name="c", subcore_axis_name="s")
    run = pl.kernel(
        _body,
        out_type=jax.ShapeDtypeStruct((NUM_OUTPUTS,), jnp.float32),
        mesh=mesh,
        scratch_types=[
            pltpu.VMEM((CARRY_PAD,), jnp.float32),
            pltpu.VMEM((CELEMS,), jnp.float32),
            pltpu.VMEM((CELEMS,), jnp.int32),
            pltpu.VMEM((CELEMS,), jnp.float32),
            pltpu.VMEM((CELEMS,), jnp.int32),
            pltpu.VMEM((N_GROUPS,), jnp.int32),
            pltpu.VMEM((NUM_OUTPUTS,), jnp.float32),
            pltpu.SemaphoreType.DMA,
            pltpu.SemaphoreType.DMA,
            pltpu.SemaphoreType.DMA,
            pltpu.SemaphoreType.DMA,
        ],
        compiler_params=pltpu.CompilerParams(needs_layout_passes=False),
    )
    group_base = (
        NUM_INPUTS + 1 + (jnp.arange(NUM_COMPUTED, dtype=jnp.int32) // 16) * 16
    )
    internal = input_ids >= group_base[:, None]
    order = jnp.argsort(internal, axis=1, stable=True)
    idx_p = jnp.take_along_axis(input_ids, order, axis=1)
    w_p = jnp.take_along_axis(W, order, axis=1)
    n_int = jnp.sum(internal.astype(jnp.int32), axis=1)
    mg = jnp.max(n_int.reshape(N_GROUPS, 16), axis=1)

    wT = w_p.reshape(-1, 16, FAN_IN).transpose(0, 2, 1).reshape(-1)
    idxT = idx_p.reshape(-1, 16, FAN_IN).transpose(0, 2, 1).reshape(-1)
    out = run(x.reshape(-1), wT, idxT, mg)
    return out[None, :]


# speedup vs baseline: 13.7081x; 1.2171x over previous
import jax
import jax.numpy as jnp
from jax import lax
from jax.experimental import pallas as pl
from jax.experimental.pallas import tpu as pltpu
from jax.experimental.pallas import tpu_sc as plsc

NUM_INPUTS = 512
NUM_COMPUTED = 4096
NUM_OUTPUTS = 128
FAN_IN = 64
SCALE = 4.9
N_UNITS = NUM_INPUTS + 1 + NUM_COMPUTED
CARRY_PAD = 4624
CHUNK = 256
N_CHUNKS = NUM_COMPUTED // CHUNK
GROUPS = CHUNK // 16
N_GROUPS = NUM_COMPUTED // 16
OUT_BASE = NUM_INPUTS + 1 + (NUM_COMPUTED - NUM_OUTPUTS)
CELEMS = CHUNK * FAN_IN


def _body(x_hbm, w_hbm, idx_hbm, mg_hbm, out_hbm,
          carry, w_a, idx_a, w_b, idx_b, mg_v, st,
          sem_wa, sem_ia, sem_wb, sem_ib):
    wid = lax.axis_index("s") * 2 + lax.axis_index("c")

    @pl.when(wid == 0)
    def _():
        lane = jnp.arange(16, dtype=jnp.int32)
        ones = jnp.ones((16,), jnp.float32)

        def start_load(c, w_buf, idx_buf, w_sem, i_sem):
            off = c * CELEMS
            pltpu.make_async_copy(
                w_hbm.at[pl.ds(off, CELEMS)], w_buf, w_sem).start()
            pltpu.make_async_copy(
                idx_hbm.at[pl.ds(off, CELEMS)], idx_buf, i_sem).start()

        def wait_load(w_buf, idx_buf, w_sem, i_sem):
            pltpu.make_async_copy(
                w_hbm.at[pl.ds(0, CELEMS)], w_buf, w_sem).wait()
            pltpu.make_async_copy(
                idx_hbm.at[pl.ds(0, CELEMS)], idx_buf, i_sem).wait()

        start_load(0, w_a, idx_a, sem_wa, sem_ia)
        pltpu.sync_copy(mg_hbm, mg_v)
        pltpu.sync_copy(x_hbm, carry.at[pl.ds(0, NUM_INPUTS)])

        def init_ones(i, _):
            carry[pl.ds(NUM_INPUTS + 16 * i, 16)] = ones
            return _

        lax.fori_loop(0, (CARRY_PAD - NUM_INPUTS) // 16, init_ones, 0)

        def make_group_step(w_v, idx_v):
            def group_step(g, state):
                pos, gg = state
                goff = g * (16 * FAN_IN)
                posv = pos + lane

                nacc = 4
                accs = [jnp.zeros((16,), jnp.float32) for _ in range(nacc)]
                for k in range(FAN_IN):
                    iv = idx_v[pl.ds(goff + 16 * k, 16)]
                    wv = w_v[pl.ds(goff + 16 * k, 16)]
                    vals = plsc.load_gather(carry, [iv])
                    wm = jnp.where(iv < pos, wv, 0.0)
                    accs[k % nacc] = accs[k % nacc] + vals * wm
                acc_ext = (accs[0] + accs[1]) + (accs[2] + accs[3])

                val = 1.0 / (1.0 + jnp.exp(-SCALE * acc_ext))
                plsc.store_scatter(carry, [posv], val)

                gbase = (gg // 16) * 16
                mgs = mg_v[pl.ds(gbase, 16)]
                m_g = jnp.sum(jnp.where(lane == gg - gbase, mgs, 0))
                tail0 = goff + 16 * FAN_IN - 16 * m_g

                def fix_body(d):
                    vcur = plsc.load_gather(carry, [posv])

                    def tail_term(j, a):
                        iv = idx_v[pl.ds(tail0 + 16 * j, 16)]
                        wv = w_v[pl.ds(tail0 + 16 * j, 16)]
                        vals = plsc.load_gather(carry, [iv])
                        vals = jnp.where(iv < posv, vals, 1.0)
                        return a + jnp.where(iv >= pos, vals * wv, 0.0)

                    acc = lax.fori_loop(0, m_g, tail_term, acc_ext)
                    vnew = 1.0 / (1.0 + jnp.exp(-SCALE * acc))
                    plsc.store_scatter(carry, [posv], vnew)
                    return jnp.sum((vnew != vcur).astype(jnp.int32))

                lax.while_loop(lambda d: d > 0, fix_body, m_g)
                return pos + 16, gg + 1

            return group_step

        step_a = make_group_step(w_a, idx_a)
        step_b = make_group_step(w_b, idx_b)

        def pair_step(p, state):
            c = 2 * p
            wait_load(w_a, idx_a, sem_wa, sem_ia)
            start_load(c + 1, w_b, idx_b, sem_wb, sem_ib)
            state = lax.fori_loop(0, GROUPS, step_a, state)
            wait_load(w_b, idx_b, sem_wb, sem_ib)

            @pl.when(p + 1 < N_CHUNKS // 2)
            def _():
                start_load(c + 2, w_a, idx_a, sem_wa, sem_ia)

            return lax.fori_loop(0, GROUPS, step_b, state)

        lax.fori_loop(0, N_CHUNKS // 2, pair_step, (NUM_INPUTS + 1, 0))

        for i in range(NUM_OUTPUTS // 16):
            iv = jnp.full((16,), OUT_BASE + 16 * i, jnp.int32) + lane
            st[pl.ds(16 * i, 16)] = plsc.load_gather(carry, [iv])
        pltpu.sync_copy(st, out_hbm)


@jax.jit
def kernel(x, W, input_ids):
    mesh = plsc.VectorSubcoreMesh(core_axis_name="c", subcore_axis_name="s")
    run = pl.kernel(
        _body,
        out_type=jax.ShapeDtypeStruct((NUM_OUTPUTS,), jnp.float32),
        mesh=mesh,
        scratch_types=[
            pltpu.VMEM((CARRY_PAD,), jnp.float32),
            pltpu.VMEM((CELEMS,), jnp.float32),
            pltpu.VMEM((CELEMS,), jnp.int32),
            pltpu.VMEM((CELEMS,), jnp.float32),
            pltpu.VMEM((CELEMS,), jnp.int32),
            pltpu.VMEM((N_GROUPS,), jnp.int32),
            pltpu.VMEM((NUM_OUTPUTS,), jnp.float32),
            pltpu.SemaphoreType.DMA,
            pltpu.SemaphoreType.DMA,
            pltpu.SemaphoreType.DMA,
            pltpu.SemaphoreType.DMA,
        ],
        compiler_params=pltpu.CompilerParams(needs_layout_passes=False),
    )
    group_base = (
        NUM_INPUTS + 1 + (jnp.arange(NUM_COMPUTED, dtype=jnp.int32) // 16) * 16
    )
    internal = input_ids >= group_base[:, None]
    _, idx_p, w_p = lax.sort(
        (internal.astype(jnp.int32), input_ids, W),
        dimension=1, num_keys=1, is_stable=True)
    n_int = jnp.sum(internal.astype(jnp.int32), axis=1)
    mg = jnp.max(n_int.reshape(N_GROUPS, 16), axis=1)

    wT = w_p.reshape(-1, 16, FAN_IN).transpose(0, 2, 1).reshape(-1)
    idxT = idx_p.reshape(-1, 16, FAN_IN).transpose(0, 2, 1).reshape(-1)
    out = run(x.reshape(-1), wT, idxT, mg)
    return out[None, :]
